# Initial kernel scaffold; baseline (speedup 1.0000x reference)
import jax
import jax.numpy as jnp
from jax.experimental import pallas as pl


def kernel(labels):
    raise NotImplementedError("write your pallas kernel here")


# speedup vs baseline: 1.2886x; 1.2886x over previous
import functools

import jax
import jax.numpy as jnp
from jax import lax
from jax.experimental import pallas as pl
from jax.experimental.pallas import tpu as pltpu
from jax.experimental.pallas import tpu_sc as plsc

_SMOOTHING = 0.1
_NUM_CLASSES = 1000
_CONFIDENCE = 1.0 - _SMOOTHING
_LOW = _SMOOTHING / _NUM_CLASSES
_HIGH = _CONFIDENCE + _LOW

_N = 16384
_NW = 32
_RPW = _N // _NW
_CH = 32
_NCH = _RPW // _CH
_L = 16


def _sc_body(labels_hbm, out_hbm, labels_v, buf0, buf1, sem0, sem1):
    c = lax.axis_index("c")
    s = lax.axis_index("s")
    wid = s * 2 + c
    base = wid * _RPW

    pltpu.sync_copy(labels_hbm.at[pl.ds(base, _RPW)], labels_v)

    low16 = jnp.full((_L,), _LOW, jnp.float32)
    high16 = jnp.full((_L,), _HIGH, jnp.float32)
    iota16 = lax.iota(jnp.int32, _L)

    ncs = _NUM_CLASSES // _L + 1

    def _init(r, _):
        for j in range(ncs):
            col = min(j * _L, _NUM_CLASSES - _L)
            buf0[r, pl.ds(col, _L)] = low16
            buf1[r, pl.ds(col, _L)] = low16
        return 0

    lax.fori_loop(0, _CH, _init, 0)

    def _patch(buf, k, val16):
        for g in range(_CH // _L):
            labs = labels_v[pl.ds(k * _CH + g * _L, _L)]
            plsc.store_scatter(buf, [iota16 + g * _L, labs], val16)

    copies = [None, None]
    for k in range(_NCH):
        buf, sem = (buf0, sem0) if k % 2 == 0 else (buf1, sem1)
        if k >= 2:
            copies[k % 2].wait()
            _patch(buf, k - 2, low16)
        _patch(buf, k, high16)
        copies[k % 2] = pltpu.async_copy(
            buf, out_hbm.at[pl.ds(base + k * _CH, _CH)], sem)
    copies[0].wait()
    copies[1].wait()


@functools.partial(jax.jit, static_argnames=())
def kernel(labels):
    run = pl.kernel(
        _sc_body,
        mesh=plsc.VectorSubcoreMesh(core_axis_name="c", subcore_axis_name="s"),
        out_type=jax.ShapeDtypeStruct((_N, _NUM_CLASSES), jnp.float32),
        scratch_types=[
            pltpu.VMEM((_RPW,), jnp.int32),
            pltpu.VMEM((_CH, _NUM_CLASSES), jnp.float32),
            pltpu.VMEM((_CH, _NUM_CLASSES), jnp.float32),
            pltpu.SemaphoreType.DMA,
            pltpu.SemaphoreType.DMA,
        ],
        compiler_params=pltpu.CompilerParams(use_tc_tiling_on_sc=False, needs_layout_passes=False),
    )
    return run(labels)


# speedup vs baseline: 2.0882x; 1.6205x over previous
import functools

import jax
import jax.numpy as jnp
from jax import lax
from jax.experimental import pallas as pl
from jax.experimental.pallas import tpu as pltpu
from jax.experimental.pallas import tpu_sc as plsc

_SMOOTHING = 0.1
_NUM_CLASSES = 1000
_CONFIDENCE = 1.0 - _SMOOTHING
_LOW = _SMOOTHING / _NUM_CLASSES
_HIGH = _CONFIDENCE + _LOW

_N = 16384
_NW = 32
_RPW = _N // _NW
_CH = 32
_NCH = _RPW // _CH
_L = 16


def _sc_body(labels_hbm, out_hbm, labels_v, buf0, buf1, sem0, sem1):
    c = lax.axis_index("c")
    s = lax.axis_index("s")
    wid = s * 2 + c
    base = wid * _RPW

    pltpu.sync_copy(labels_hbm.at[pl.ds(base, _RPW)], labels_v)

    low16 = jnp.full((_L,), _LOW, jnp.float32)
    high16 = jnp.full((_L,), _HIGH, jnp.float32)
    iota16 = lax.iota(jnp.int32, _L)

    ncs = _NUM_CLASSES // _L + 1

    def _init(r, _):
        for j in range(ncs):
            col = min(j * _L, _NUM_CLASSES - _L)
            buf0[r, pl.ds(col, _L)] = low16
            buf1[r, pl.ds(col, _L)] = low16
        return 0

    lax.fori_loop(0, _CH, _init, 0)

    def _patch(buf, k, val16):
        for g in range(_CH // _L):
            labs = labels_v[pl.ds(k * _CH + g * _L, _L)]
            plsc.store_scatter(buf, [iota16 + g * _L, labs], val16)

    copies = [None, None]
    for k in range(_NCH):
        buf, sem = (buf0, sem0) if k % 2 == 0 else (buf1, sem1)
        if k >= 2:
            copies[k % 2].wait()
            _patch(buf, k - 2, low16)
        _patch(buf, k, high16)
        copies[k % 2] = pltpu.async_copy(
            buf, out_hbm.at[pl.ds(base + k * _CH, _CH)], sem)
    copies[0].wait()
    copies[1].wait()


@functools.partial(jax.jit, static_argnames=())
def kernel(labels):
    run = pl.kernel(
        _sc_body,
        mesh=plsc.VectorSubcoreMesh(core_axis_name="c", subcore_axis_name="s"),
        out_type=jax.ShapeDtypeStruct((_N, _NUM_CLASSES), jnp.float32),
        scratch_types=[
            pltpu.VMEM((_RPW,), jnp.int32),
            pltpu.VMEM((_CH, _NUM_CLASSES), jnp.float32),
            pltpu.VMEM((_CH, _NUM_CLASSES), jnp.float32),
            pltpu.SemaphoreType.DMA,
            pltpu.SemaphoreType.DMA,
        ],
        compiler_params=pltpu.CompilerParams(use_tc_tiling_on_sc=True, needs_layout_passes=False),
    )
    return run(labels)


# speedup vs baseline: 2.5645x; 1.2281x over previous
import jax
import jax.numpy as jnp
from jax.experimental import pallas as pl
from jax.experimental.pallas import tpu as pltpu

_SMOOTHING = 0.1
_NUM_CLASSES = 1000
_CONFIDENCE = 1.0 - _SMOOTHING
_LOW = _SMOOTHING / _NUM_CLASSES
_HIGH = _CONFIDENCE + _LOW

_BR = 4096


def _smooth_kernel(lab_ref, out_ref):
    lab = lab_ref[0, 0, :]
    cols = jax.lax.broadcasted_iota(jnp.int32, (_BR, _NUM_CLASSES), 1)
    hit = cols == lab[:, None]
    out_ref[...] = jnp.where(hit, _HIGH, _LOW).astype(jnp.float32)


def kernel(labels):
    n = labels.shape[0]
    nb = n // _BR
    lab3 = labels.reshape(nb, 1, _BR)
    return pl.pallas_call(
        _smooth_kernel,
        grid=(nb,),
        in_specs=[pl.BlockSpec((1, 1, _BR), lambda i: (i, 0, 0))],
        out_specs=pl.BlockSpec((_BR, _NUM_CLASSES), lambda i: (i, 0)),
        out_shape=jax.ShapeDtypeStruct((n, _NUM_CLASSES), jnp.float32),
        compiler_params=pltpu.CompilerParams(
            dimension_semantics=("parallel",),
        ),
    )(lab3)


# speedup vs baseline: 2.6050x; 1.0158x over previous
import jax
import jax.numpy as jnp
from jax.experimental import pallas as pl
from jax.experimental.pallas import tpu as pltpu

_SMOOTHING = 0.1
_NUM_CLASSES = 1000
_CONFIDENCE = 1.0 - _SMOOTHING
_LOW = _SMOOTHING / _NUM_CLASSES
_HIGH = _CONFIDENCE + _LOW

_N = 16384
_CH = 512
_NCH = _N // _CH
_NBUF = 4


def _smooth_kernel(lab_ref, out_ref, b0, b1, b2, b3, s0, s1, s2, s3):
    bufs = (b0, b1, b2, b3)
    sems = (s0, s1, s2, s3)
    cols = jax.lax.broadcasted_iota(jnp.int32, (_CH, _NUM_CLASSES), 1)
    copies = [None] * _NBUF
    for k in range(_NCH):
        b = k % _NBUF
        if copies[b] is not None:
            copies[b].wait()
        lab = lab_ref[pl.ds(k * _CH, _CH)]
        bufs[b][...] = jnp.where(cols == lab[:, None], _HIGH, _LOW).astype(
            jnp.float32)
        copies[b] = pltpu.make_async_copy(
            bufs[b], out_ref.at[pl.ds(k * _CH, _CH)], sems[b])
        copies[b].start()
    for b in range(_NBUF):
        copies[b].wait()


def kernel(labels):
    return pl.pallas_call(
        _smooth_kernel,
        in_specs=[pl.BlockSpec(memory_space=pltpu.VMEM)],
        out_specs=pl.BlockSpec(memory_space=pl.ANY),
        out_shape=jax.ShapeDtypeStruct((_N, _NUM_CLASSES), jnp.float32),
        scratch_shapes=[pltpu.VMEM((_CH, _NUM_CLASSES), jnp.float32)] * _NBUF
        + [pltpu.SemaphoreType.DMA] * _NBUF,
    )(labels)


# speedup vs baseline: 9.6719x; 3.7129x over previous
import jax
import jax.numpy as jnp
from jax.experimental import pallas as pl

_SMOOTHING = 0.1
_NUM_CLASSES = 1000
_CONFIDENCE = 1.0 - _SMOOTHING
_LOW = _SMOOTHING / _NUM_CLASSES
_HIGH = _CONFIDENCE + _LOW

_BC = 2048


def _smooth_kernel(lab_ref, out_ref):
    lab = lab_ref[0, 0, :]
    rows = jax.lax.broadcasted_iota(jnp.int32, (_NUM_CLASSES, _BC), 0)
    hit = rows == lab[None, :]
    out_ref[...] = jnp.where(hit, _HIGH, _LOW).astype(jnp.float32)


def kernel(labels):
    n = labels.shape[0]
    nb = n // _BC
    lab3 = labels.reshape(nb, 1, _BC)
    out_t = pl.pallas_call(
        _smooth_kernel,
        grid=(nb,),
        in_specs=[pl.BlockSpec((1, 1, _BC), lambda i: (i, 0, 0))],
        out_specs=pl.BlockSpec((_NUM_CLASSES, _BC), lambda i: (0, i)),
        out_shape=jax.ShapeDtypeStruct((_NUM_CLASSES, n), jnp.float32),
    )(lab3)
    return out_t.T
